# baseline (speedup 1.0000x reference)
import functools

import jax
import jax.numpy as jnp
from jax import lax
from jax.experimental import pallas as pl
from jax.experimental.pallas import tpu as pltpu
from jax.experimental.pallas import tpu_sc as plsc


def kernel(tokenized_prompts, token_embedding, ctx):
    n_cls, ctx_len = tokenized_prompts.shape
    _, d = token_embedding.shape
    n_ctx = ctx.shape[0]

    NC, NS = 2, 16
    NW = NC * NS
    chunk = (-(-n_cls // NW) + 7) // 8 * 8
    lin = (ctx_len // 8) * 8
    ntail = ctx_len - lin

    mesh = plsc.VectorSubcoreMesh(
        core_axis_name="c", subcore_axis_name="s", num_cores=NC, num_subcores=NS
    )

    @functools.partial(
        pl.kernel,
        out_type=jax.ShapeDtypeStruct((n_cls, ctx_len, d), jnp.float32),
        mesh=mesh,
        scratch_types=[
            pltpu.VMEM((chunk, ctx_len), jnp.int32),
            pltpu.VMEM((2, ctx_len, d), jnp.float32),
            pltpu.VMEM((n_ctx, d), jnp.float32),
            pltpu.VMEM((ntail,), jnp.int32),
            pltpu.SemaphoreType.DMA,
            pltpu.SemaphoreType.DMA,
        ],
    )
    def _k(tp_hbm, table_hbm, ctx_hbm, tid_hbm, out_hbm, idx_v, rows_v, ctx_v, rid_v, sem, sem2):
        w = lax.axis_index("s") * NC + lax.axis_index("c")
        start = w * chunk
        cnt = jnp.clip(n_cls - start, 0, chunk)
        stage = jnp.minimum(start, n_cls - chunk)
        off = start - stage
        pltpu.sync_copy(ctx_hbm, ctx_v)
        pltpu.sync_copy(tp_hbm.at[pl.ds(stage, chunk)], idx_v)
        pltpu.sync_copy(tid_hbm, rid_v)

        def gather(j, b):
            return pltpu.make_async_copy(
                table_hbm.at[idx_v.at[off + j]], rows_v.at[b], sem
            )

        def store_head(j, b):
            return pltpu.make_async_copy(
                rows_v.at[b, pl.ds(0, lin)], out_hbm.at[start + j, pl.ds(0, lin)], sem2
            )

        def store_tail(j, b):
            return pltpu.make_async_copy(
                rows_v.at[b, pl.ds(lin, ntail)], out_hbm.at[start + j].at[rid_v], sem2
            )

        @pl.when(cnt > 0)
        def _run():
            gather(0, 0).start()

            def body(j, carry):
                b = lax.rem(j, 2)

                @pl.when(j > 0)
                def _():
                    store_head(j - 1, 1 - b).wait()
                    store_tail(j - 1, 1 - b).wait()

                gather(j, b).wait()

                @pl.when(j + 1 < cnt)
                def _():
                    gather(j + 1, 1 - b).start()

                for r in range(n_ctx):
                    for k in range(d // 16):
                        rows_v[b, 1 + r, pl.ds(16 * k, 16)] = ctx_v[r, pl.ds(16 * k, 16)]
                store_head(j, b).start()
                store_tail(j, b).start()
                return carry

            lax.fori_loop(0, cnt, body, 0)
            store_head(cnt - 1, lax.rem(cnt - 1, 2)).wait()
            store_tail(cnt - 1, lax.rem(cnt - 1, 2)).wait()

    tail_ids = jnp.arange(lin, ctx_len, dtype=jnp.int32)
    return _k(tokenized_prompts, token_embedding, ctx, tail_ids)


# speedup vs baseline: 1.7753x; 1.7753x over previous
import functools

import jax
import jax.numpy as jnp
from jax import lax
from jax.experimental import pallas as pl
from jax.experimental.pallas import tpu as pltpu
from jax.experimental.pallas import tpu_sc as plsc


def kernel(tokenized_prompts, token_embedding, ctx):
    n_cls, ctx_len = tokenized_prompts.shape
    _, d = token_embedding.shape
    n_ctx = ctx.shape[0]

    NC, NS = 2, 16
    NW = NC * NS
    chunk = (-(-n_cls // NW) + 7) // 8 * 8

    mesh = plsc.VectorSubcoreMesh(
        core_axis_name="c", subcore_axis_name="s", num_cores=NC, num_subcores=NS
    )

    @functools.partial(
        pl.kernel,
        out_type=jax.ShapeDtypeStruct((ctx_len * n_cls, d), jnp.float32),
        mesh=mesh,
        scratch_types=[
            pltpu.VMEM((chunk, ctx_len), jnp.int32),
            pltpu.VMEM((3, ctx_len, d), jnp.float32),
            pltpu.VMEM((n_ctx, d), jnp.float32),
            pltpu.VMEM((3, ctx_len), jnp.int32),
            pltpu.SemaphoreType.DMA((3,)),
            pltpu.SemaphoreType.DMA((3,)),
        ],
    )
    def _k(tp_hbm, table_hbm, ctx_hbm, out_hbm, idx_v, rows_v, ctx_v, rid_v, sem, sem2):
        w = lax.axis_index("s") * NC + lax.axis_index("c")
        start = w * chunk
        cnt = jnp.clip(n_cls - start, 0, chunk)
        stage = jnp.minimum(start, n_cls - chunk)
        off = start - stage
        pltpu.sync_copy(ctx_hbm, ctx_v)
        pltpu.sync_copy(tp_hbm.at[pl.ds(stage, chunk)], idx_v)

        iota16 = lax.iota(jnp.int32, 16)

        def gather(j, b):
            return pltpu.make_async_copy(
                table_hbm.at[idx_v.at[off + j]], rows_v.at[b], sem.at[b]
            )

        def scatter(j, b):
            return pltpu.make_async_copy(
                rows_v.at[b], out_hbm.at[rid_v.at[b]], sem2.at[b]
            )

        @pl.when(cnt > 0)
        def _run():
            gather(0, 0).start()

            @pl.when(cnt > 1)
            def _():
                gather(1, 1).start()

            def body(j, carry):
                b = lax.rem(j, 3)
                c_abs = start + j
                gather(j, b).wait()
                for r in range(n_ctx):
                    for k in range(d // 16):
                        rows_v[b, 1 + r, pl.ds(16 * k, 16)] = ctx_v[r, pl.ds(16 * k, 16)]
                for k0 in range(0, ctx_len - 16, 16):
                    rid_v[b, pl.ds(k0, 16)] = (k0 + iota16) * n_cls + c_abs
                k0 = ctx_len - 16
                rid_v[b, pl.ds(k0, 16)] = (k0 + iota16) * n_cls + c_abs
                scatter(j, b).start()

                @pl.when(j + 2 < cnt)
                def _():
                    bn = lax.rem(j + 2, 3)

                    @pl.when(j > 0)
                    def _():
                        scatter(j - 1, bn).wait()

                    gather(j + 2, bn).start()

                return carry

            lax.fori_loop(0, cnt, body, 0)

            @pl.when(cnt > 2)
            def _():
                scatter(cnt - 3, lax.rem(cnt - 3, 3)).wait()

            @pl.when(cnt > 1)
            def _():
                scatter(cnt - 2, lax.rem(cnt - 2, 3)).wait()

            scatter(cnt - 1, lax.rem(cnt - 1, 3)).wait()

    res = _k(tokenized_prompts, token_embedding, ctx)
    return res.reshape(ctx_len, n_cls, d).transpose(1, 0, 2)
